# initial kernel scaffold (unmeasured)
import jax
import jax.numpy as jnp
from jax import lax
from jax.experimental import pallas as pl
from jax.experimental.pallas import tpu as pltpu

N_DEV = 16
B_PER = 2
SQ = 128
D = 512
H_PER = 8
DH = 64
SCALE = 0.125


def kernel(x, Wq, Wo, Wk, Wv):
    def body(x_ref, wq_ref, wo_ref, wk_ref, wv_ref, out_ref,
             xg_ref, part_ref, rs_send_ref, rs_recv_ref,
             wq16, wk16, wv16, wo16,
             ag_send_sems, ag_recv_sems, rs_send_sems, rs_recv_sems):
        my = lax.axis_index("i")
        left = lax.rem(my - 1 + N_DEV, N_DEV)
        right = lax.rem(my + 1, N_DEV)

        barrier = pltpu.get_barrier_semaphore()
        for nbr in (left, right):
            pl.semaphore_signal(barrier, inc=1, device_id=(nbr,),
                                device_id_type=pl.DeviceIdType.MESH)
        pl.semaphore_wait(barrier, 2)

        wq16[...] = wq_ref[...].astype(jnp.bfloat16)
        wk16[...] = wk_ref[...].astype(jnp.bfloat16)
        wv16[...] = wv_ref[...].astype(jnp.bfloat16)
        wo16[...] = wo_ref[...].astype(jnp.bfloat16)

        xg_ref[0] = x_ref[...].astype(jnp.bfloat16)

        for h in range(N_DEV - 1):
            rdma = pltpu.make_async_remote_copy(
                src_ref=xg_ref.at[h],
                dst_ref=xg_ref.at[h + 1],
                send_sem=ag_send_sems.at[h],
                recv_sem=ag_recv_sems.at[h],
                device_id=(right,),
                device_id_type=pl.DeviceIdType.MESH,
            )
            rdma.start()
            rdma.wait()

        def compute_slot(r, carry):
            for b in range(B_PER):
                xb = xg_ref[r, b]
                q = jnp.dot(xb, wq16[...],
                            preferred_element_type=jnp.float32).astype(jnp.bfloat16)
                k = jnp.dot(xb, wk16[...],
                            preferred_element_type=jnp.float32).astype(jnp.bfloat16)
                v = jnp.dot(xb, wv16[...],
                            preferred_element_type=jnp.float32).astype(jnp.bfloat16)
                o_cols = []
                for hh in range(H_PER):
                    sl = slice(hh * DH, (hh + 1) * DH)
                    qh, kh, vh = q[:, sl], k[:, sl], v[:, sl]
                    s = lax.dot_general(
                        qh, kh, (((1,), (1,)), ((), ())),
                        preferred_element_type=jnp.float32) * SCALE
                    m = jnp.max(s, axis=1, keepdims=True)
                    e = jnp.exp(s - m)
                    pmat = (e / jnp.sum(e, axis=1, keepdims=True)).astype(jnp.bfloat16)
                    o_cols.append(jnp.dot(pmat, vh,
                                          preferred_element_type=jnp.float32))
                attn = jnp.concatenate(o_cols, axis=1).astype(jnp.bfloat16)
                part_ref[r, b] = jnp.dot(attn, wo16[...],
                                         preferred_element_type=jnp.float32)
            return carry
        lax.fori_loop(0, N_DEV, compute_slot, 0)

        for t in range(N_DEV - 1):
            if t == 0:
                rs_send_ref[0] = part_ref[1]
            else:
                rs_send_ref[t] = part_ref[t + 1] + rs_recv_ref[t - 1]
            rdma = pltpu.make_async_remote_copy(
                src_ref=rs_send_ref.at[t],
                dst_ref=rs_recv_ref.at[t],
                send_sem=rs_send_sems.at[t],
                recv_sem=rs_recv_sems.at[t],
                device_id=(right,),
                device_id_type=pl.DeviceIdType.MESH,
            )
            rdma.start()
            rdma.wait()

        out_ref[...] = part_ref[0] + rs_recv_ref[N_DEV - 2]

    return pl.pallas_call(
        body,
        out_shape=jax.ShapeDtypeStruct((B_PER, SQ, D), jnp.float32),
        in_specs=[pl.BlockSpec(memory_space=pltpu.VMEM)] * 5,
        out_specs=pl.BlockSpec(memory_space=pltpu.VMEM),
        scratch_shapes=[
            pltpu.VMEM((N_DEV, B_PER, SQ, D), jnp.bfloat16),
            pltpu.VMEM((N_DEV, B_PER, SQ, D), jnp.float32),
            pltpu.VMEM((N_DEV - 1, B_PER, SQ, D), jnp.float32),
            pltpu.VMEM((N_DEV - 1, B_PER, SQ, D), jnp.float32),
            pltpu.VMEM((D, D), jnp.bfloat16),
            pltpu.VMEM((D, D), jnp.bfloat16),
            pltpu.VMEM((D, D), jnp.bfloat16),
            pltpu.VMEM((D, D), jnp.bfloat16),
            pltpu.SemaphoreType.DMA((N_DEV - 1,)),
            pltpu.SemaphoreType.DMA((N_DEV - 1,)),
            pltpu.SemaphoreType.DMA((N_DEV - 1,)),
            pltpu.SemaphoreType.DMA((N_DEV - 1,)),
        ],
        compiler_params=pltpu.CompilerParams(collective_id=0),
    )(x, Wq, Wk, Wv, Wo)


def _kernel_body_order_note():
    pass


# baseline (device time: 292851 ns/iter reference)
import jax
import jax.numpy as jnp
from jax import lax
from jax.experimental import pallas as pl
from jax.experimental.pallas import tpu as pltpu

N_DEV = 16
B_PER = 2
SQ = 128
D = 512
H_PER = 8
DH = 64
SCALE = 0.125


def kernel(x, Wq, Wo, Wk, Wv):
    def body(x_ref, wq_ref, wk_ref, wv_ref, wo_ref, out_ref,
             xg_ref, part_ref, rs_send_ref, rs_recv_ref,
             wq16, wk16, wv16, wo16,
             ag_send_sems, ag_recv_sems, rs_send_sems, rs_recv_sems):
        my = lax.axis_index("i")
        left = lax.rem(my - 1 + N_DEV, N_DEV)
        right = lax.rem(my + 1, N_DEV)

        barrier = pltpu.get_barrier_semaphore()
        for nbr in (left, right):
            pl.semaphore_signal(barrier, inc=1, device_id=(nbr,),
                                device_id_type=pl.DeviceIdType.MESH)
        pl.semaphore_wait(barrier, 2)

        wq16[...] = wq_ref[...].astype(jnp.bfloat16)
        wk16[...] = wk_ref[...].astype(jnp.bfloat16)
        wv16[...] = wv_ref[...].astype(jnp.bfloat16)
        wo16[...] = wo_ref[...].astype(jnp.bfloat16)

        xg_ref[0] = x_ref[...].astype(jnp.bfloat16)

        for h in range(N_DEV - 1):
            rdma = pltpu.make_async_remote_copy(
                src_ref=xg_ref.at[h],
                dst_ref=xg_ref.at[h + 1],
                send_sem=ag_send_sems.at[h],
                recv_sem=ag_recv_sems.at[h],
                device_id=(right,),
                device_id_type=pl.DeviceIdType.MESH,
            )
            rdma.start()
            rdma.wait()

        def compute_slot(r, carry):
            for b in range(B_PER):
                xb = xg_ref[r, b]
                q = jnp.dot(xb, wq16[...],
                            preferred_element_type=jnp.float32).astype(jnp.bfloat16)
                k = jnp.dot(xb, wk16[...],
                            preferred_element_type=jnp.float32).astype(jnp.bfloat16)
                v = jnp.dot(xb, wv16[...],
                            preferred_element_type=jnp.float32).astype(jnp.bfloat16)
                o_cols = []
                for hh in range(H_PER):
                    sl = slice(hh * DH, (hh + 1) * DH)
                    qh, kh, vh = q[:, sl], k[:, sl], v[:, sl]
                    s = lax.dot_general(
                        qh, kh, (((1,), (1,)), ((), ())),
                        preferred_element_type=jnp.float32) * SCALE
                    m = jnp.max(s, axis=1, keepdims=True)
                    e = jnp.exp(s - m)
                    pmat = (e / jnp.sum(e, axis=1, keepdims=True)).astype(jnp.bfloat16)
                    o_cols.append(jnp.dot(pmat, vh,
                                          preferred_element_type=jnp.float32))
                attn = jnp.concatenate(o_cols, axis=1).astype(jnp.bfloat16)
                part_ref[r, b] = jnp.dot(attn, wo16[...],
                                         preferred_element_type=jnp.float32)
            return carry
        lax.fori_loop(0, N_DEV, compute_slot, 0)

        for t in range(N_DEV - 1):
            if t == 0:
                rs_send_ref[0] = part_ref[1]
            else:
                rs_send_ref[t] = part_ref[t + 1] + rs_recv_ref[t - 1]
            rdma = pltpu.make_async_remote_copy(
                src_ref=rs_send_ref.at[t],
                dst_ref=rs_recv_ref.at[t],
                send_sem=rs_send_sems.at[t],
                recv_sem=rs_recv_sems.at[t],
                device_id=(right,),
                device_id_type=pl.DeviceIdType.MESH,
            )
            rdma.start()
            rdma.wait()

        out_ref[...] = part_ref[0] + rs_recv_ref[N_DEV - 2]

    return pl.pallas_call(
        body,
        out_shape=jax.ShapeDtypeStruct((B_PER, SQ, D), jnp.float32),
        in_specs=[pl.BlockSpec(memory_space=pltpu.VMEM)] * 5,
        out_specs=pl.BlockSpec(memory_space=pltpu.VMEM),
        scratch_shapes=[
            pltpu.VMEM((N_DEV, B_PER, SQ, D), jnp.bfloat16),
            pltpu.VMEM((N_DEV, B_PER, SQ, D), jnp.float32),
            pltpu.VMEM((N_DEV - 1, B_PER, SQ, D), jnp.float32),
            pltpu.VMEM((N_DEV - 1, B_PER, SQ, D), jnp.float32),
            pltpu.VMEM((D, D), jnp.bfloat16),
            pltpu.VMEM((D, D), jnp.bfloat16),
            pltpu.VMEM((D, D), jnp.bfloat16),
            pltpu.VMEM((D, D), jnp.bfloat16),
            pltpu.SemaphoreType.DMA((N_DEV - 1,)),
            pltpu.SemaphoreType.DMA((N_DEV - 1,)),
            pltpu.SemaphoreType.DMA((N_DEV - 1,)),
            pltpu.SemaphoreType.DMA((N_DEV - 1,)),
        ],
        compiler_params=pltpu.CompilerParams(collective_id=0),
    )(x, Wq, Wk, Wv, Wo)


# device time: 123924 ns/iter; 2.3631x vs baseline; 2.3631x over previous
import jax
import jax.numpy as jnp
from jax import lax
from jax.experimental import pallas as pl
from jax.experimental.pallas import tpu as pltpu

N_DEV = 16
B_PER = 2
SQ = 128
D = 512
H_PER = 8
DH = 64
SCALE = 0.125


def kernel(x, Wq, Wo, Wk, Wv):
    def body(x_ref, wq_ref, wk_ref, wv_ref, wo_ref, out_ref,
             xg_ref, part_ref, rs_send_ref, rs_recv_ref,
             wq16, wk16, wv16, wo16,
             ag_send_sems, ag_recv_sems, rs_send_sems, rs_recv_sems):
        my = lax.axis_index("i")
        left = lax.rem(my - 1 + N_DEV, N_DEV)
        right = lax.rem(my + 1, N_DEV)

        barrier = pltpu.get_barrier_semaphore()
        for nbr in (left, right):
            pl.semaphore_signal(barrier, inc=1, device_id=(nbr,),
                                device_id_type=pl.DeviceIdType.MESH)
        pl.semaphore_wait(barrier, 2)

        def ag_desc(h):
            return pltpu.make_async_remote_copy(
                src_ref=xg_ref.at[h],
                dst_ref=xg_ref.at[h + 1],
                send_sem=ag_send_sems.at[h],
                recv_sem=ag_recv_sems.at[h],
                device_id=(right,),
                device_id_type=pl.DeviceIdType.MESH,
            )

        def rs_desc(t):
            return pltpu.make_async_remote_copy(
                src_ref=rs_send_ref.at[t],
                dst_ref=rs_recv_ref.at[t],
                send_sem=rs_send_sems.at[t],
                recv_sem=rs_recv_sems.at[t],
                device_id=(right,),
                device_id_type=pl.DeviceIdType.MESH,
            )

        def compute_slot(r):
            xb2 = xg_ref[r].reshape(B_PER * SQ, D)
            q = jnp.dot(xb2, wq16[...],
                        preferred_element_type=jnp.float32).astype(jnp.bfloat16)
            k = jnp.dot(xb2, wk16[...],
                        preferred_element_type=jnp.float32).astype(jnp.bfloat16)
            v = jnp.dot(xb2, wv16[...],
                        preferred_element_type=jnp.float32).astype(jnp.bfloat16)
            o_rows = []
            for b in range(B_PER):
                rsl = slice(b * SQ, (b + 1) * SQ)
                o_cols = []
                for hh in range(H_PER):
                    csl = slice(hh * DH, (hh + 1) * DH)
                    qh, kh, vh = q[rsl, csl], k[rsl, csl], v[rsl, csl]
                    s = lax.dot_general(
                        qh, kh, (((1,), (1,)), ((), ())),
                        preferred_element_type=jnp.float32) * SCALE
                    m = jnp.max(s, axis=1, keepdims=True)
                    e = jnp.exp(s - m)
                    pmat = (e / jnp.sum(e, axis=1, keepdims=True)).astype(jnp.bfloat16)
                    o_cols.append(jnp.dot(pmat, vh,
                                          preferred_element_type=jnp.float32))
                o_rows.append(jnp.concatenate(o_cols, axis=1))
            attn = jnp.concatenate(o_rows, axis=0).astype(jnp.bfloat16)
            part_ref[r] = jnp.dot(attn, wo16[...],
                                  preferred_element_type=jnp.float32
                                  ).reshape(B_PER, SQ, D)

        def rs_step(t):
            if t == 0:
                rs_send_ref[0] = part_ref[1].astype(jnp.bfloat16)
            else:
                rs_descs[t - 1].wait_recv()
                rs_send_ref[t] = (part_ref[t + 1]
                                  + rs_recv_ref[t - 1].astype(jnp.float32)
                                  ).astype(jnp.bfloat16)
            rs_descs[t].start()

        ag_descs = [ag_desc(h) for h in range(N_DEV - 1)]
        rs_descs = [rs_desc(t) for t in range(N_DEV - 1)]

        wq16[...] = wq_ref[...].astype(jnp.bfloat16)
        wk16[...] = wk_ref[...].astype(jnp.bfloat16)
        wv16[...] = wv_ref[...].astype(jnp.bfloat16)
        wo16[...] = wo_ref[...].astype(jnp.bfloat16)
        xg_ref[0] = x_ref[...].astype(jnp.bfloat16)

        ag_descs[0].start()
        compute_slot(0)
        for h in range(1, N_DEV):
            ag_descs[h - 1].wait_recv()
            if h < N_DEV - 1:
                ag_descs[h].start()
            compute_slot(h)
            rs_step(h - 1)

        rs_descs[N_DEV - 2].wait_recv()
        out_ref[...] = part_ref[0] + rs_recv_ref[N_DEV - 2].astype(jnp.float32)

        for h in range(N_DEV - 1):
            ag_descs[h].wait_send()
            rs_descs[h].wait_send()

    return pl.pallas_call(
        body,
        out_shape=jax.ShapeDtypeStruct((B_PER, SQ, D), jnp.float32),
        in_specs=[pl.BlockSpec(memory_space=pltpu.VMEM)] * 5,
        out_specs=pl.BlockSpec(memory_space=pltpu.VMEM),
        scratch_shapes=[
            pltpu.VMEM((N_DEV, B_PER, SQ, D), jnp.bfloat16),
            pltpu.VMEM((N_DEV, B_PER, SQ, D), jnp.float32),
            pltpu.VMEM((N_DEV - 1, B_PER, SQ, D), jnp.bfloat16),
            pltpu.VMEM((N_DEV - 1, B_PER, SQ, D), jnp.bfloat16),
            pltpu.VMEM((D, D), jnp.bfloat16),
            pltpu.VMEM((D, D), jnp.bfloat16),
            pltpu.VMEM((D, D), jnp.bfloat16),
            pltpu.VMEM((D, D), jnp.bfloat16),
            pltpu.SemaphoreType.DMA((N_DEV - 1,)),
            pltpu.SemaphoreType.DMA((N_DEV - 1,)),
            pltpu.SemaphoreType.DMA((N_DEV - 1,)),
            pltpu.SemaphoreType.DMA((N_DEV - 1,)),
        ],
        compiler_params=pltpu.CompilerParams(collective_id=0),
    )(x, Wq, Wk, Wv, Wo)


# device time: 122317 ns/iter; 2.3942x vs baseline; 1.0131x over previous
import jax
import jax.numpy as jnp
from jax import lax
from jax.experimental import pallas as pl
from jax.experimental.pallas import tpu as pltpu

N_DEV = 16
B_PER = 2
SQ = 128
D = 512
H_PER = 8
DH = 64
SCALE = 0.125

RING = [0, 1, 2, 3, 7, 6, 5, 9, 10, 11, 15, 14, 13, 12, 8, 4]
POS = [0] * N_DEV
for _p, _m in enumerate(RING):
    POS[_m] = _p


def kernel(x, Wq, Wo, Wk, Wv):
    def body(x_ref, wq_ref, wk_ref, wv_ref, wo_ref, out_ref,
             xg_ref, part_ref, rs_send_ref, rs_recv_ref,
             wq16, wk16, wv16, wo16,
             ag_send_sems, ag_recv_sems, rs_send_sems, rs_recv_sems):
        def lookup(table, idx):
            val = jnp.int32(table[0])
            for p in range(1, N_DEV):
                val = jnp.where(idx == p, jnp.int32(table[p]), val)
            return val

        my = lax.axis_index("i")
        pos = lookup(POS, my)
        left = lookup(RING, lax.rem(pos - 1 + N_DEV, N_DEV))
        right = lookup(RING, lax.rem(pos + 1, N_DEV))

        barrier = pltpu.get_barrier_semaphore()
        for nbr in (left, right):
            pl.semaphore_signal(barrier, inc=1, device_id=(nbr,),
                                device_id_type=pl.DeviceIdType.MESH)
        pl.semaphore_wait(barrier, 2)

        def ag_desc(h):
            return pltpu.make_async_remote_copy(
                src_ref=xg_ref.at[h],
                dst_ref=xg_ref.at[h + 1],
                send_sem=ag_send_sems.at[h],
                recv_sem=ag_recv_sems.at[h],
                device_id=(right,),
                device_id_type=pl.DeviceIdType.MESH,
            )

        def rs_desc(t):
            return pltpu.make_async_remote_copy(
                src_ref=rs_send_ref.at[t],
                dst_ref=rs_recv_ref.at[t],
                send_sem=rs_send_sems.at[t],
                recv_sem=rs_recv_sems.at[t],
                device_id=(right,),
                device_id_type=pl.DeviceIdType.MESH,
            )

        def compute_slot(r):
            xb2 = xg_ref[r].reshape(B_PER * SQ, D)
            q = jnp.dot(xb2, wq16[...],
                        preferred_element_type=jnp.float32).astype(jnp.bfloat16)
            k = jnp.dot(xb2, wk16[...],
                        preferred_element_type=jnp.float32).astype(jnp.bfloat16)
            v = jnp.dot(xb2, wv16[...],
                        preferred_element_type=jnp.float32).astype(jnp.bfloat16)
            o_rows = []
            for b in range(B_PER):
                rsl = slice(b * SQ, (b + 1) * SQ)
                o_cols = []
                for hh in range(H_PER):
                    csl = slice(hh * DH, (hh + 1) * DH)
                    qh, kh, vh = q[rsl, csl], k[rsl, csl], v[rsl, csl]
                    s = lax.dot_general(
                        qh, kh, (((1,), (1,)), ((), ())),
                        preferred_element_type=jnp.float32) * SCALE
                    m = jnp.max(s, axis=1, keepdims=True)
                    e = jnp.exp(s - m)
                    pmat = (e / jnp.sum(e, axis=1, keepdims=True)).astype(jnp.bfloat16)
                    o_cols.append(jnp.dot(pmat, vh,
                                          preferred_element_type=jnp.float32))
                o_rows.append(jnp.concatenate(o_cols, axis=1))
            attn = jnp.concatenate(o_rows, axis=0).astype(jnp.bfloat16)
            part_ref[r] = jnp.dot(attn, wo16[...],
                                  preferred_element_type=jnp.float32
                                  ).reshape(B_PER, SQ, D)

        def rs_step(t):
            if t == 0:
                rs_send_ref[0] = part_ref[1].astype(jnp.bfloat16)
            else:
                rs_descs[t - 1].wait_recv()
                rs_send_ref[t] = (part_ref[t + 1]
                                  + rs_recv_ref[t - 1].astype(jnp.float32)
                                  ).astype(jnp.bfloat16)
            rs_descs[t].start()

        ag_descs = [ag_desc(h) for h in range(N_DEV - 1)]
        rs_descs = [rs_desc(t) for t in range(N_DEV - 1)]

        wq16[...] = wq_ref[...].astype(jnp.bfloat16)
        wk16[...] = wk_ref[...].astype(jnp.bfloat16)
        wv16[...] = wv_ref[...].astype(jnp.bfloat16)
        wo16[...] = wo_ref[...].astype(jnp.bfloat16)
        xg_ref[0] = x_ref[...].astype(jnp.bfloat16)

        ag_descs[0].start()
        compute_slot(0)
        for h in range(1, N_DEV):
            ag_descs[h - 1].wait_recv()
            if h < N_DEV - 1:
                ag_descs[h].start()
            compute_slot(h)
            rs_step(h - 1)

        rs_descs[N_DEV - 2].wait_recv()
        out_ref[...] = part_ref[0] + rs_recv_ref[N_DEV - 2].astype(jnp.float32)

        for h in range(N_DEV - 1):
            ag_descs[h].wait_send()
            rs_descs[h].wait_send()

    return pl.pallas_call(
        body,
        out_shape=jax.ShapeDtypeStruct((B_PER, SQ, D), jnp.float32),
        in_specs=[pl.BlockSpec(memory_space=pltpu.VMEM)] * 5,
        out_specs=pl.BlockSpec(memory_space=pltpu.VMEM),
        scratch_shapes=[
            pltpu.VMEM((N_DEV, B_PER, SQ, D), jnp.bfloat16),
            pltpu.VMEM((N_DEV, B_PER, SQ, D), jnp.float32),
            pltpu.VMEM((N_DEV - 1, B_PER, SQ, D), jnp.bfloat16),
            pltpu.VMEM((N_DEV - 1, B_PER, SQ, D), jnp.bfloat16),
            pltpu.VMEM((D, D), jnp.bfloat16),
            pltpu.VMEM((D, D), jnp.bfloat16),
            pltpu.VMEM((D, D), jnp.bfloat16),
            pltpu.VMEM((D, D), jnp.bfloat16),
            pltpu.SemaphoreType.DMA((N_DEV - 1,)),
            pltpu.SemaphoreType.DMA((N_DEV - 1,)),
            pltpu.SemaphoreType.DMA((N_DEV - 1,)),
            pltpu.SemaphoreType.DMA((N_DEV - 1,)),
        ],
        compiler_params=pltpu.CompilerParams(collective_id=0),
    )(x, Wq, Wk, Wv, Wo)
